# Initial kernel scaffold; baseline (speedup 1.0000x reference)
import jax
import jax.numpy as jnp
from jax.experimental import pallas as pl


def kernel(x, edge_index, batch, W1, b1, W2, b2):
    raise NotImplementedError("write your pallas kernel here")


# speedup vs baseline: 48.4378x; 48.4378x over previous
import functools

import jax
import jax.numpy as jnp
from jax import lax
from jax.experimental import pallas as pl
from jax.experimental.pallas import tpu as pltpu
from jax.experimental.pallas import tpu_sc as plsc

N = 10000
E = 320000
F_IN = 128
H = 16

NC, NS, L = 2, 16, 16
NW = NC * NS
CH = 125
NCHUNK = E // CH
PW = NCHUNK // NW
NP = 10240
SLC = NP // NS


def _mesh():
    return plsc.VectorSubcoreMesh(
        core_axis_name="c", subcore_axis_name="s",
        num_cores=NC, num_subcores=NS)


_SC_PARAMS = pltpu.CompilerParams(use_tc_tiling_on_sc=False)


def _worker(c, s):
    return s * NC + c


def _sc_count(dst2d):
    @functools.partial(
        pl.kernel,
        out_type=jax.ShapeDtypeStruct((NC, NP), jnp.float32),
        mesh=_mesh(),
        compiler_params=_SC_PARAMS,
        scratch_types=[
            pltpu.VMEM((PW, CH), jnp.int32),
            pltpu.VMEM((128,), jnp.float32),
            pltpu.VMEM((SLC,), jnp.float32),
            pltpu.VMEM_SHARED((NP,), jnp.float32),
        ],
    )
    def k(dst_hbm, out_hbm, dst_v, ones_v, zero_v, acc):
        c = lax.axis_index("c")
        s = lax.axis_index("s")
        w = _worker(c, s)

        def fill(i, _):
            ones_v[pl.ds(i * L, L)] = jnp.ones((L,), jnp.float32)
            zero_v[pl.ds(i * L, L)] = jnp.zeros((L,), jnp.float32)
            return 0
        lax.fori_loop(0, 128 // L, fill, 0)

        def fillz(i, _):
            zero_v[pl.ds(128 + i * L, L)] = jnp.zeros((L,), jnp.float32)
            return 0
        lax.fori_loop(0, (SLC - 128) // L, fillz, 0)

        pltpu.sync_copy(zero_v, acc.at[pl.ds(s * SLC, SLC)])
        pltpu.sync_copy(dst_hbm.at[pl.ds(w * PW, PW)], dst_v)
        plsc.subcore_barrier()

        def chunk(j, _):
            pltpu.sync_copy(ones_v.at[pl.ds(0, CH)],
                            acc.at[dst_v.at[j]], add=True)
            return 0
        lax.fori_loop(0, PW, chunk, 0)

        plsc.subcore_barrier()
        pltpu.sync_copy(acc.at[pl.ds(s * SLC, SLC)],
                        out_hbm.at[c, pl.ds(s * SLC, SLC)])

    return k(dst2d)


def _sc_agg16(src2d, dst2d, hs):
    @functools.partial(
        pl.kernel,
        out_type=jax.ShapeDtypeStruct((NC, NP, H), jnp.float32),
        mesh=_mesh(),
        compiler_params=_SC_PARAMS,
        scratch_types=[
            pltpu.VMEM((PW, CH), jnp.int32),
            pltpu.VMEM((PW, CH), jnp.int32),
            pltpu.VMEM((CH, H), jnp.float32),
            pltpu.VMEM((CH, H), jnp.float32),
            pltpu.VMEM((64, H), jnp.float32),
            pltpu.VMEM_SHARED((NP, H), jnp.float32),
            pltpu.SemaphoreType.DMA,
            pltpu.SemaphoreType.DMA,
        ],
    )
    def k(src_hbm, dst_hbm, hs_hbm, out_hbm,
          src_v, dst_v, buf0, buf1, zero_v, acc, sem0, sem1):
        c = lax.axis_index("c")
        s = lax.axis_index("s")
        w = _worker(c, s)

        def fillz(i, _):
            zero_v[i, :] = jnp.zeros((H,), jnp.float32)
            return 0
        lax.fori_loop(0, 64, fillz, 0)

        def zrow(i, _):
            pltpu.sync_copy(zero_v, acc.at[pl.ds(s * SLC + i * 64, 64), :])
            return 0
        lax.fori_loop(0, SLC // 64, zrow, 0)

        pltpu.sync_copy(src_hbm.at[pl.ds(w * PW, PW)], src_v)
        pltpu.sync_copy(dst_hbm.at[pl.ds(w * PW, PW)], dst_v)
        plsc.subcore_barrier()

        pltpu.async_copy(hs_hbm.at[src_v.at[0]], buf0, sem0)

        def pair(i, _):
            j0 = i * 2
            pltpu.async_copy(hs_hbm.at[src_v.at[j0 + 1]], buf1, sem1)
            pltpu.make_async_copy(hs_hbm.at[src_v.at[j0]], buf0, sem0).wait()
            pltpu.sync_copy(buf0, acc.at[dst_v.at[j0]], add=True)

            @pl.when(i < PW // 2 - 1)
            def _():
                pltpu.async_copy(hs_hbm.at[src_v.at[j0 + 2]], buf0, sem0)

            pltpu.make_async_copy(
                hs_hbm.at[src_v.at[j0 + 1]], buf1, sem1).wait()
            pltpu.sync_copy(buf1, acc.at[dst_v.at[j0 + 1]], add=True)
            return 0
        lax.fori_loop(0, PW // 2, pair, 0)

        plsc.subcore_barrier()
        pltpu.sync_copy(acc.at[pl.ds(s * SLC, SLC), :],
                        out_hbm.at[c, pl.ds(s * SLC, SLC), :])

    return k(src2d, dst2d, hs)


def _sc_agg1(src2d, dst2d, gs):
    @functools.partial(
        pl.kernel,
        out_type=jax.ShapeDtypeStruct((NC, NP), jnp.float32),
        mesh=_mesh(),
        compiler_params=_SC_PARAMS,
        scratch_types=[
            pltpu.VMEM((PW, CH), jnp.int32),
            pltpu.VMEM((PW, CH), jnp.int32),
            pltpu.VMEM((128,), jnp.float32),
            pltpu.VMEM((128,), jnp.float32),
            pltpu.VMEM((SLC,), jnp.float32),
            pltpu.VMEM_SHARED((NP,), jnp.float32),
            pltpu.SemaphoreType.DMA,
            pltpu.SemaphoreType.DMA,
        ],
    )
    def k(src_hbm, dst_hbm, gs_hbm, out_hbm,
          src_v, dst_v, buf0, buf1, zero_v, acc, sem0, sem1):
        c = lax.axis_index("c")
        s = lax.axis_index("s")
        w = _worker(c, s)

        def fillz(i, _):
            zero_v[pl.ds(i * L, L)] = jnp.zeros((L,), jnp.float32)
            return 0
        lax.fori_loop(0, SLC // L, fillz, 0)

        pltpu.sync_copy(zero_v, acc.at[pl.ds(s * SLC, SLC)])
        pltpu.sync_copy(src_hbm.at[pl.ds(w * PW, PW)], src_v)
        pltpu.sync_copy(dst_hbm.at[pl.ds(w * PW, PW)], dst_v)
        plsc.subcore_barrier()

        pltpu.async_copy(gs_hbm.at[src_v.at[0]], buf0.at[pl.ds(0, CH)], sem0)

        def pair(i, _):
            j0 = i * 2
            pltpu.async_copy(gs_hbm.at[src_v.at[j0 + 1]],
                             buf1.at[pl.ds(0, CH)], sem1)
            pltpu.make_async_copy(gs_hbm.at[src_v.at[j0]],
                                  buf0.at[pl.ds(0, CH)], sem0).wait()
            pltpu.sync_copy(buf0.at[pl.ds(0, CH)],
                            acc.at[dst_v.at[j0]], add=True)

            @pl.when(i < PW // 2 - 1)
            def _():
                pltpu.async_copy(gs_hbm.at[src_v.at[j0 + 2]],
                                 buf0.at[pl.ds(0, CH)], sem0)

            pltpu.make_async_copy(gs_hbm.at[src_v.at[j0 + 1]],
                                  buf1.at[pl.ds(0, CH)], sem1).wait()
            pltpu.sync_copy(buf1.at[pl.ds(0, CH)],
                            acc.at[dst_v.at[j0 + 1]], add=True)
            return 0
        lax.fori_loop(0, PW // 2, pair, 0)

        plsc.subcore_barrier()
        pltpu.sync_copy(acc.at[pl.ds(s * SLC, SLC)],
                        out_hbm.at[c, pl.ds(s * SLC, SLC)])

    return k(src2d, dst2d, gs)


NB = 2000


def _tc_prep_body(c0_ref, c1_ref, x_ref, w1_ref, dis_ref, hs_ref):
    deg = c0_ref[...] + c1_ref[...] + 1.0
    dis = lax.rsqrt(deg)
    h = jnp.dot(x_ref[...], w1_ref[...], preferred_element_type=jnp.float32)
    dis_ref[...] = dis
    hs_ref[...] = h * dis


def _tc_prep(c0, c1, x, W1):
    return pl.pallas_call(
        _tc_prep_body,
        grid=(N // NB,),
        in_specs=[
            pl.BlockSpec((NB, 1), lambda i: (i, 0)),
            pl.BlockSpec((NB, 1), lambda i: (i, 0)),
            pl.BlockSpec((NB, F_IN), lambda i: (i, 0)),
            pl.BlockSpec((F_IN, H), lambda i: (0, 0)),
        ],
        out_specs=[
            pl.BlockSpec((NB, 1), lambda i: (i, 0)),
            pl.BlockSpec((NB, H), lambda i: (i, 0)),
        ],
        out_shape=[
            jax.ShapeDtypeStruct((N, 1), jnp.float32),
            jax.ShapeDtypeStruct((N, H), jnp.float32),
        ],
    )(c0, c1, x, W1)


def _tc_mid_body(a0_ref, a1_ref, dis_ref, hs_ref, b1_ref, w2_ref, gs_ref):
    a = a0_ref[...] + a1_ref[...] + hs_ref[...]
    t = jnp.maximum(dis_ref[...] * a + b1_ref[...], 0.0)
    g = jnp.sum(t * w2_ref[...], axis=1, keepdims=True)
    gs_ref[...] = g * dis_ref[...]


def _tc_mid(a0, a1, dis, hs, b1row, w2row):
    return pl.pallas_call(
        _tc_mid_body,
        grid=(N // NB,),
        in_specs=[
            pl.BlockSpec((NB, H), lambda i: (i, 0)),
            pl.BlockSpec((NB, H), lambda i: (i, 0)),
            pl.BlockSpec((NB, 1), lambda i: (i, 0)),
            pl.BlockSpec((NB, H), lambda i: (i, 0)),
            pl.BlockSpec((1, H), lambda i: (0, 0)),
            pl.BlockSpec((1, H), lambda i: (0, 0)),
        ],
        out_specs=pl.BlockSpec((NB, 1), lambda i: (i, 0)),
        out_shape=jax.ShapeDtypeStruct((N, 1), jnp.float32),
    )(a0, a1, dis, hs, b1row, w2row)


def _tc_out_body(a0_ref, a1_ref, dis_ref, gs_ref, b2_ref, bsz_ref, out_ref):
    a = a0_ref[...] + a1_ref[...] + gs_ref[...]
    out_ref[...] = (dis_ref[...] * a + b2_ref[...]) * bsz_ref[...]


def _tc_out(a0, a1, dis, gs, b2v, bszv):
    return pl.pallas_call(
        _tc_out_body,
        grid=(N // NB,),
        in_specs=[
            pl.BlockSpec((NB, 1), lambda i: (i, 0)),
            pl.BlockSpec((NB, 1), lambda i: (i, 0)),
            pl.BlockSpec((NB, 1), lambda i: (i, 0)),
            pl.BlockSpec((NB, 1), lambda i: (i, 0)),
            pl.BlockSpec((1, 1), lambda i: (0, 0)),
            pl.BlockSpec((1, 1), lambda i: (0, 0)),
        ],
        out_specs=pl.BlockSpec((NB, 1), lambda i: (i, 0)),
        out_shape=jax.ShapeDtypeStruct((N, 1), jnp.float32),
    )(a0, a1, dis, gs, b2v, bszv)


def kernel(x, edge_index, batch, W1, b1, W2, b2):
    src2d = edge_index[0].reshape(NCHUNK, CH)
    dst2d = edge_index[1].reshape(NCHUNK, CH)

    count = _sc_count(dst2d)
    c0 = count[0, :N].reshape(N, 1)
    c1 = count[1, :N].reshape(N, 1)
    dis, hs1 = _tc_prep(c0, c1, x, W1)

    agg1 = _sc_agg16(src2d, dst2d, hs1)
    gs = _tc_mid(agg1[0, :N, :], agg1[1, :N, :], dis, hs1,
                 b1.reshape(1, H), W2.reshape(1, H))

    agg2 = _sc_agg1(src2d, dst2d, gs.reshape(N))
    bszv = (batch[-1] + 1).astype(jnp.float32).reshape(1, 1)
    out = _tc_out(agg2[0, :N].reshape(N, 1), agg2[1, :N].reshape(N, 1),
                  dis, gs, b2.reshape(1, 1), bszv)
    return out.reshape(1, N)


# speedup vs baseline: 67.3116x; 1.3896x over previous
import functools

import jax
import jax.numpy as jnp
from jax import lax
from jax.experimental import pallas as pl
from jax.experimental.pallas import tpu as pltpu
from jax.experimental.pallas import tpu_sc as plsc

N = 10000
E = 320000
F_IN = 128
H = 16

NC, NS, L = 2, 16, 16
NW = NC * NS
EW = E // NW
NSEG = 4
SEG = EW // NSEG
NP = 10240
SLC = NP // NS


def _mesh():
    return plsc.VectorSubcoreMesh(
        core_axis_name="c", subcore_axis_name="s",
        num_cores=NC, num_subcores=NS)


_SC_PARAMS = pltpu.CompilerParams(use_tc_tiling_on_sc=False,
                                  needs_layout_passes=False)


def _worker(c, s):
    return s * NC + c


def _sc_count(dst1d):
    @functools.partial(
        pl.kernel,
        out_type=jax.ShapeDtypeStruct((NC, NP), jnp.float32),
        mesh=_mesh(),
        compiler_params=_SC_PARAMS,
        scratch_types=[
            pltpu.VMEM((EW,), jnp.int32),
            pltpu.VMEM((EW,), jnp.float32),
            pltpu.VMEM((SLC,), jnp.float32),
            pltpu.VMEM_SHARED((NP,), jnp.float32),
        ],
    )
    def k(dst_hbm, out_hbm, dst_v, ones_v, zero_v, acc):
        c = lax.axis_index("c")
        s = lax.axis_index("s")
        w = _worker(c, s)

        def fillo(i, _):
            ones_v[pl.ds(i * L, L)] = jnp.ones((L,), jnp.float32)
            return 0
        lax.fori_loop(0, EW // L, fillo, 0)

        def fillz(i, _):
            zero_v[pl.ds(i * L, L)] = jnp.zeros((L,), jnp.float32)
            return 0
        lax.fori_loop(0, SLC // L, fillz, 0)

        pltpu.sync_copy(zero_v, acc.at[pl.ds(s * SLC, SLC)])
        pltpu.sync_copy(dst_hbm.at[pl.ds(w * EW, EW)], dst_v)
        plsc.subcore_barrier()

        pltpu.sync_copy(ones_v, acc.at[dst_v], add=True)

        plsc.subcore_barrier()
        pltpu.sync_copy(acc.at[pl.ds(s * SLC, SLC)],
                        out_hbm.at[c, pl.ds(s * SLC, SLC)])

    return k(dst1d)


def _sc_agg16(src2d, dst2d, hs):
    @functools.partial(
        pl.kernel,
        out_type=jax.ShapeDtypeStruct((NC, NP, H), jnp.float32),
        mesh=_mesh(),
        compiler_params=_SC_PARAMS,
        scratch_types=[
            pltpu.VMEM((NSEG, SEG), jnp.int32),
            pltpu.VMEM((NSEG, SEG), jnp.int32),
            pltpu.VMEM((SEG, H), jnp.float32),
            pltpu.VMEM((SEG, H), jnp.float32),
            pltpu.VMEM((64, H), jnp.float32),
            pltpu.VMEM_SHARED((NP, H), jnp.float32),
            pltpu.SemaphoreType.DMA,
            pltpu.SemaphoreType.DMA,
        ],
    )
    def k(src_hbm, dst_hbm, hs_hbm, out_hbm,
          src_v, dst_v, buf0, buf1, zero_v, acc, sem0, sem1):
        c = lax.axis_index("c")
        s = lax.axis_index("s")
        w = _worker(c, s)

        def fillz(i, _):
            zero_v[i, :] = jnp.zeros((H,), jnp.float32)
            return 0
        lax.fori_loop(0, 64, fillz, 0)

        def zrow(i, _):
            pltpu.sync_copy(zero_v, acc.at[pl.ds(s * SLC + i * 64, 64), :])
            return 0
        lax.fori_loop(0, SLC // 64, zrow, 0)

        pltpu.sync_copy(src_hbm.at[pl.ds(w * NSEG, NSEG)], src_v)
        pltpu.sync_copy(dst_hbm.at[pl.ds(w * NSEG, NSEG)], dst_v)
        plsc.subcore_barrier()

        bufs = (buf0, buf1)
        sems = (sem0, sem1)
        pltpu.async_copy(hs_hbm.at[src_v.at[0]], buf0, sem0)
        for j in range(NSEG):
            b = j % 2
            if j + 1 < NSEG:
                pltpu.async_copy(hs_hbm.at[src_v.at[j + 1]],
                                 bufs[1 - b], sems[1 - b])
            pltpu.make_async_copy(hs_hbm.at[src_v.at[j]],
                                  bufs[b], sems[b]).wait()
            pltpu.sync_copy(bufs[b], acc.at[dst_v.at[j]], add=True)

        plsc.subcore_barrier()
        pltpu.sync_copy(acc.at[pl.ds(s * SLC, SLC), :],
                        out_hbm.at[c, pl.ds(s * SLC, SLC), :])

    return k(src2d, dst2d, hs)


def _sc_agg1(src1d, dst1d, gs):
    @functools.partial(
        pl.kernel,
        out_type=jax.ShapeDtypeStruct((NC, NP), jnp.float32),
        mesh=_mesh(),
        compiler_params=_SC_PARAMS,
        scratch_types=[
            pltpu.VMEM((EW,), jnp.int32),
            pltpu.VMEM((EW,), jnp.int32),
            pltpu.VMEM((EW,), jnp.float32),
            pltpu.VMEM((N,), jnp.float32),
            pltpu.VMEM((SLC,), jnp.float32),
            pltpu.VMEM_SHARED((NP,), jnp.float32),
        ],
    )
    def k(src_hbm, dst_hbm, gs_hbm, out_hbm,
          src_v, dst_v, msg_v, gs_v, zero_v, acc):
        c = lax.axis_index("c")
        s = lax.axis_index("s")
        w = _worker(c, s)

        def fillz(i, _):
            zero_v[pl.ds(i * L, L)] = jnp.zeros((L,), jnp.float32)
            return 0
        lax.fori_loop(0, SLC // L, fillz, 0)

        pltpu.sync_copy(zero_v, acc.at[pl.ds(s * SLC, SLC)])
        pltpu.sync_copy(src_hbm.at[pl.ds(w * EW, EW)], src_v)
        pltpu.sync_copy(dst_hbm.at[pl.ds(w * EW, EW)], dst_v)
        pltpu.sync_copy(gs_hbm, gs_v)
        plsc.subcore_barrier()

        def gat(i, _):
            idx = src_v[pl.ds(i * L, L)]
            msg_v[pl.ds(i * L, L)] = plsc.load_gather(gs_v, [idx])
            return 0
        lax.fori_loop(0, EW // L, gat, 0)

        pltpu.sync_copy(msg_v, acc.at[dst_v], add=True)

        plsc.subcore_barrier()
        pltpu.sync_copy(acc.at[pl.ds(s * SLC, SLC)],
                        out_hbm.at[c, pl.ds(s * SLC, SLC)])

    return k(src1d, dst1d, gs)


NB = 2000


def _tc_prep_body(c0_ref, c1_ref, x_ref, w1_ref, dis_ref, hs_ref):
    deg = c0_ref[...] + c1_ref[...] + 1.0
    dis = lax.rsqrt(deg)
    h = jnp.dot(x_ref[...], w1_ref[...], preferred_element_type=jnp.float32)
    dis_ref[...] = dis
    hs_ref[...] = h * dis


def _tc_prep(c0, c1, x, W1):
    return pl.pallas_call(
        _tc_prep_body,
        grid=(N // NB,),
        in_specs=[
            pl.BlockSpec((NB, 1), lambda i: (i, 0)),
            pl.BlockSpec((NB, 1), lambda i: (i, 0)),
            pl.BlockSpec((NB, F_IN), lambda i: (i, 0)),
            pl.BlockSpec((F_IN, H), lambda i: (0, 0)),
        ],
        out_specs=[
            pl.BlockSpec((NB, 1), lambda i: (i, 0)),
            pl.BlockSpec((NB, H), lambda i: (i, 0)),
        ],
        out_shape=[
            jax.ShapeDtypeStruct((N, 1), jnp.float32),
            jax.ShapeDtypeStruct((N, H), jnp.float32),
        ],
    )(c0, c1, x, W1)


def _tc_mid_body(a0_ref, a1_ref, dis_ref, hs_ref, b1_ref, w2_ref, gs_ref):
    a = a0_ref[...] + a1_ref[...] + hs_ref[...]
    t = jnp.maximum(dis_ref[...] * a + b1_ref[...], 0.0)
    g = jnp.sum(t * w2_ref[...], axis=1, keepdims=True)
    gs_ref[...] = g * dis_ref[...]


def _tc_mid(a0, a1, dis, hs, b1row, w2row):
    return pl.pallas_call(
        _tc_mid_body,
        grid=(N // NB,),
        in_specs=[
            pl.BlockSpec((NB, H), lambda i: (i, 0)),
            pl.BlockSpec((NB, H), lambda i: (i, 0)),
            pl.BlockSpec((NB, 1), lambda i: (i, 0)),
            pl.BlockSpec((NB, H), lambda i: (i, 0)),
            pl.BlockSpec((1, H), lambda i: (0, 0)),
            pl.BlockSpec((1, H), lambda i: (0, 0)),
        ],
        out_specs=pl.BlockSpec((NB, 1), lambda i: (i, 0)),
        out_shape=jax.ShapeDtypeStruct((N, 1), jnp.float32),
    )(a0, a1, dis, hs, b1row, w2row)


def _tc_out_body(a0_ref, a1_ref, dis_ref, gs_ref, b2_ref, bsz_ref, out_ref):
    a = a0_ref[...] + a1_ref[...] + gs_ref[...]
    out_ref[...] = (dis_ref[...] * a + b2_ref[...]) * bsz_ref[...]


def _tc_out(a0, a1, dis, gs, b2v, bszv):
    return pl.pallas_call(
        _tc_out_body,
        grid=(N // NB,),
        in_specs=[
            pl.BlockSpec((NB, 1), lambda i: (i, 0)),
            pl.BlockSpec((NB, 1), lambda i: (i, 0)),
            pl.BlockSpec((NB, 1), lambda i: (i, 0)),
            pl.BlockSpec((NB, 1), lambda i: (i, 0)),
            pl.BlockSpec((1, 1), lambda i: (0, 0)),
            pl.BlockSpec((1, 1), lambda i: (0, 0)),
        ],
        out_specs=pl.BlockSpec((NB, 1), lambda i: (i, 0)),
        out_shape=jax.ShapeDtypeStruct((N, 1), jnp.float32),
    )(a0, a1, dis, gs, b2v, bszv)


def kernel(x, edge_index, batch, W1, b1, W2, b2):
    src1d = edge_index[0]
    dst1d = edge_index[1]
    src2d = src1d.reshape(NW * NSEG, SEG)
    dst2d = dst1d.reshape(NW * NSEG, SEG)

    count = _sc_count(dst1d)
    c0 = count[0, :N].reshape(N, 1)
    c1 = count[1, :N].reshape(N, 1)
    dis, hs1 = _tc_prep(c0, c1, x, W1)

    agg1 = _sc_agg16(src2d, dst2d, hs1)
    gs = _tc_mid(agg1[0, :N, :], agg1[1, :N, :], dis, hs1,
                 b1.reshape(1, H), W2.reshape(1, H))

    agg2 = _sc_agg1(src1d, dst1d, gs.reshape(N))
    bszv = (batch[-1] + 1).astype(jnp.float32).reshape(1, 1)
    out = _tc_out(agg2[0, :N].reshape(N, 1), agg2[1, :N].reshape(N, 1),
                  dis, gs, b2.reshape(1, 1), bszv)
    return out.reshape(1, N)


# speedup vs baseline: 90.5107x; 1.3447x over previous
import functools

import jax
import jax.numpy as jnp
from jax import lax
from jax.experimental import pallas as pl
from jax.experimental.pallas import tpu as pltpu
from jax.experimental.pallas import tpu_sc as plsc

N = 10000
E = 320000
F_IN = 128
H = 16

NC, NS, L = 2, 16, 16
NW = NC * NS
EW = E // NW
NSEG = 5
SEG = EW // NSEG
NP = 10240
SLC = NP // NS


def _mesh():
    return plsc.VectorSubcoreMesh(
        core_axis_name="c", subcore_axis_name="s",
        num_cores=NC, num_subcores=NS)


_SC_PARAMS = pltpu.CompilerParams(use_tc_tiling_on_sc=False,
                                  needs_layout_passes=False)


def _worker(c, s):
    return s * NC + c


def _sc_count(edge_index):
    @functools.partial(
        pl.kernel,
        out_type=jax.ShapeDtypeStruct((NC, NP), jnp.float32),
        mesh=_mesh(),
        compiler_params=_SC_PARAMS,
        scratch_types=[
            pltpu.VMEM((EW,), jnp.int32),
            pltpu.VMEM((EW,), jnp.float32),
            pltpu.VMEM((SLC,), jnp.float32),
            pltpu.VMEM_SHARED((NP,), jnp.float32),
        ],
    )
    def k(edge_hbm, out_hbm, dst_v, ones_v, zero_v, acc):
        c = lax.axis_index("c")
        s = lax.axis_index("s")
        w = _worker(c, s)

        def fillo(i, _):
            ones_v[pl.ds(i * L, L)] = jnp.ones((L,), jnp.float32)
            return 0
        lax.fori_loop(0, EW // L, fillo, 0)

        def fillz(i, _):
            zero_v[pl.ds(i * L, L)] = jnp.zeros((L,), jnp.float32)
            return 0
        lax.fori_loop(0, SLC // L, fillz, 0)

        pltpu.sync_copy(zero_v, acc.at[pl.ds(s * SLC, SLC)])
        pltpu.sync_copy(edge_hbm.at[1, pl.ds(w * EW, EW)], dst_v)
        plsc.subcore_barrier()

        pltpu.sync_copy(ones_v, acc.at[dst_v], add=True)

        plsc.subcore_barrier()
        pltpu.sync_copy(acc.at[pl.ds(s * SLC, SLC)],
                        out_hbm.at[c, pl.ds(s * SLC, SLC)])

    return k(edge_index)


def _sc_agg16(edge_index, hs):
    @functools.partial(
        pl.kernel,
        out_type=jax.ShapeDtypeStruct((NC, NP, H), jnp.float32),
        mesh=_mesh(),
        compiler_params=_SC_PARAMS,
        scratch_types=[
            pltpu.VMEM((NSEG, SEG), jnp.int32),
            pltpu.VMEM((NSEG, SEG), jnp.int32),
            pltpu.VMEM((SEG, H), jnp.float32),
            pltpu.VMEM((SEG, H), jnp.float32),
            pltpu.VMEM((64, H), jnp.float32),
            pltpu.VMEM_SHARED((NP, H), jnp.float32),
            pltpu.SemaphoreType.DMA,
            pltpu.SemaphoreType.DMA,
        ],
    )
    def k(edge_hbm, hs_hbm, out_hbm,
          src_v, dst_v, buf0, buf1, zero_v, acc, sem0, sem1):
        c = lax.axis_index("c")
        s = lax.axis_index("s")
        w = _worker(c, s)

        def fillz(i, _):
            zero_v[i, :] = jnp.zeros((H,), jnp.float32)
            return 0
        lax.fori_loop(0, 64, fillz, 0)

        def zrow(i, _):
            pltpu.sync_copy(zero_v, acc.at[pl.ds(s * SLC + i * 64, 64), :])
            return 0
        lax.fori_loop(0, SLC // 64, zrow, 0)

        for j in range(NSEG):
            pltpu.sync_copy(edge_hbm.at[0, pl.ds(w * EW + j * SEG, SEG)],
                            src_v.at[j])
            pltpu.sync_copy(edge_hbm.at[1, pl.ds(w * EW + j * SEG, SEG)],
                            dst_v.at[j])
        plsc.subcore_barrier()

        bufs = (buf0, buf1)
        sems = (sem0, sem1)
        pltpu.async_copy(hs_hbm.at[src_v.at[0]], buf0, sem0)
        for j in range(NSEG):
            b = j % 2
            if j + 1 < NSEG:
                pltpu.async_copy(hs_hbm.at[src_v.at[j + 1]],
                                 bufs[1 - b], sems[1 - b])
            pltpu.make_async_copy(hs_hbm.at[src_v.at[j]],
                                  bufs[b], sems[b]).wait()
            pltpu.sync_copy(bufs[b], acc.at[dst_v.at[j]], add=True)

        plsc.subcore_barrier()
        pltpu.sync_copy(acc.at[pl.ds(s * SLC, SLC), :],
                        out_hbm.at[c, pl.ds(s * SLC, SLC), :])

    return k(edge_index, hs)


def _sc_agg1(edge_index, gs):
    @functools.partial(
        pl.kernel,
        out_type=jax.ShapeDtypeStruct((NC, NP), jnp.float32),
        mesh=_mesh(),
        compiler_params=_SC_PARAMS,
        scratch_types=[
            pltpu.VMEM((EW,), jnp.int32),
            pltpu.VMEM((EW,), jnp.int32),
            pltpu.VMEM((EW,), jnp.float32),
            pltpu.VMEM((N,), jnp.float32),
            pltpu.VMEM((SLC,), jnp.float32),
            pltpu.VMEM_SHARED((NP,), jnp.float32),
        ],
    )
    def k(edge_hbm, gs_hbm, out_hbm,
          src_v, dst_v, msg_v, gs_v, zero_v, acc):
        c = lax.axis_index("c")
        s = lax.axis_index("s")
        w = _worker(c, s)

        def fillz(i, _):
            zero_v[pl.ds(i * L, L)] = jnp.zeros((L,), jnp.float32)
            return 0
        lax.fori_loop(0, SLC // L, fillz, 0)

        pltpu.sync_copy(zero_v, acc.at[pl.ds(s * SLC, SLC)])
        pltpu.sync_copy(edge_hbm.at[0, pl.ds(w * EW, EW)], src_v)
        pltpu.sync_copy(edge_hbm.at[1, pl.ds(w * EW, EW)], dst_v)
        pltpu.sync_copy(gs_hbm, gs_v)
        plsc.subcore_barrier()

        def gat(i, _):
            idx = src_v[pl.ds(i * L, L)]
            msg_v[pl.ds(i * L, L)] = plsc.load_gather(gs_v, [idx])
            return 0
        lax.fori_loop(0, EW // L, gat, 0)

        pltpu.sync_copy(msg_v, acc.at[dst_v], add=True)

        plsc.subcore_barrier()
        pltpu.sync_copy(acc.at[pl.ds(s * SLC, SLC)],
                        out_hbm.at[c, pl.ds(s * SLC, SLC)])

    return k(edge_index, gs)


def _tc_prep_body(cnt_ref, x_ref, w1_ref, dis_ref, hs_ref):
    deg = cnt_ref[0:1, :N] + cnt_ref[1:2, :N] + 1.0
    dis = jnp.transpose(lax.rsqrt(deg))
    h = jnp.dot(x_ref[...], w1_ref[...], preferred_element_type=jnp.float32)
    dis_ref[...] = dis
    hs_ref[...] = h * dis


def _tc_prep(cnt, x, W1):
    return pl.pallas_call(
        _tc_prep_body,
        out_shape=[
            jax.ShapeDtypeStruct((N, 1), jnp.float32),
            jax.ShapeDtypeStruct((N, H), jnp.float32),
        ],
    )(cnt, x, W1)


def _tc_mid_body(agg_ref, dis_ref, hs_ref, b1_ref, w2_ref, gs_ref):
    a = agg_ref[0, :N, :] + agg_ref[1, :N, :] + hs_ref[...]
    t = jnp.maximum(dis_ref[...] * a + b1_ref[...], 0.0)
    g = jnp.sum(t * w2_ref[...], axis=1, keepdims=True)
    gs_ref[...] = g * dis_ref[...]


def _tc_mid(agg, dis, hs, b1row, w2row):
    return pl.pallas_call(
        _tc_mid_body,
        out_shape=jax.ShapeDtypeStruct((N, 1), jnp.float32),
    )(agg, dis, hs, b1row, w2row)


def _tc_out_body(a2_ref, dis_ref, gs_ref, batch_ref, b2_ref, out_ref):
    bsz = (batch_ref[0, N - 1] + 1).astype(jnp.float32)
    a = a2_ref[0:1, :N] + a2_ref[1:2, :N] + gs_ref[...]
    out_ref[...] = (dis_ref[...] * a + b2_ref[0, 0]) * bsz


def _tc_out(a2, dis_row, gs_row, batch_row, b2v):
    return pl.pallas_call(
        _tc_out_body,
        out_shape=jax.ShapeDtypeStruct((1, N), jnp.float32),
    )(a2, dis_row, gs_row, batch_row, b2v)


def kernel(x, edge_index, batch, W1, b1, W2, b2):
    count = _sc_count(edge_index)
    dis, hs1 = _tc_prep(count, x, W1)

    agg1 = _sc_agg16(edge_index, hs1)
    gs = _tc_mid(agg1, dis, hs1,
                 b1.reshape(1, H), W2.reshape(1, H))

    agg2 = _sc_agg1(edge_index, gs.reshape(N))
    out = _tc_out(agg2, dis.reshape(1, N), gs.reshape(1, N),
                  batch.reshape(1, N), b2.reshape(1, 1))
    return out


# speedup vs baseline: 103.2411x; 1.1407x over previous
import functools

import jax
import jax.numpy as jnp
from jax import lax
from jax.experimental import pallas as pl
from jax.experimental.pallas import tpu as pltpu
from jax.experimental.pallas import tpu_sc as plsc

N = 10000
E = 320000
F_IN = 128
H = 16

NC, NS, L = 2, 16, 16
NW = NC * NS
EW = E // NW
NSEG = 10
SEG = EW // NSEG
NBUF = 3
NP = 10240
SLC = NP // NS


def _mesh():
    return plsc.VectorSubcoreMesh(
        core_axis_name="c", subcore_axis_name="s",
        num_cores=NC, num_subcores=NS)


_SC_PARAMS = pltpu.CompilerParams(use_tc_tiling_on_sc=False,
                                  needs_layout_passes=False)


def _worker(c, s):
    return s * NC + c


def _sc_count(edge_index):
    @functools.partial(
        pl.kernel,
        out_type=jax.ShapeDtypeStruct((NC, NP), jnp.float32),
        mesh=_mesh(),
        compiler_params=_SC_PARAMS,
        scratch_types=[
            pltpu.VMEM((EW,), jnp.int32),
            pltpu.VMEM((EW,), jnp.float32),
            pltpu.VMEM((SLC,), jnp.float32),
            pltpu.VMEM_SHARED((NP,), jnp.float32),
            pltpu.SemaphoreType.DMA,
        ],
    )
    def k(edge_hbm, out_hbm, dst_v, ones_v, zero_v, acc, semi):
        c = lax.axis_index("c")
        s = lax.axis_index("s")
        w = _worker(c, s)

        cp = pltpu.async_copy(edge_hbm.at[1, pl.ds(w * EW, EW)], dst_v, semi)

        def fillo(i, _):
            ones_v[pl.ds(i * L, L)] = jnp.ones((L,), jnp.float32)
            return 0
        lax.fori_loop(0, EW // L, fillo, 0)

        def fillz(i, _):
            zero_v[pl.ds(i * L, L)] = jnp.zeros((L,), jnp.float32)
            return 0
        lax.fori_loop(0, SLC // L, fillz, 0)

        @pl.when(c == 0)
        def _():
            pltpu.sync_copy(ones_v.at[pl.ds(0, SLC)],
                            acc.at[pl.ds(s * SLC, SLC)])

        @pl.when(c != 0)
        def _():
            pltpu.sync_copy(zero_v, acc.at[pl.ds(s * SLC, SLC)])

        cp.wait()
        plsc.subcore_barrier()

        pltpu.sync_copy(ones_v, acc.at[dst_v], add=True)

        plsc.subcore_barrier()
        pltpu.sync_copy(acc.at[pl.ds(s * SLC, SLC)],
                        out_hbm.at[c, pl.ds(s * SLC, SLC)])

    return k(edge_index)


def _sc_agg16(edge_index, hs):
    @functools.partial(
        pl.kernel,
        out_type=jax.ShapeDtypeStruct((NC, NP, H), jnp.float32),
        mesh=_mesh(),
        compiler_params=_SC_PARAMS,
        scratch_types=[
            pltpu.VMEM((EW,), jnp.int32),
            pltpu.VMEM((NSEG, SEG), jnp.int32),
            pltpu.VMEM((SEG, H), jnp.float32),
            pltpu.VMEM((SEG, H), jnp.float32),
            pltpu.VMEM((SEG, H), jnp.float32),
            pltpu.VMEM((64, H), jnp.float32),
            pltpu.VMEM_SHARED((NP, H), jnp.float32),
            pltpu.SemaphoreType.DMA,
            pltpu.SemaphoreType.DMA,
            pltpu.SemaphoreType.DMA,
            pltpu.SemaphoreType.DMA,
        ],
    )
    def k(edge_hbm, hs_hbm, out_hbm,
          src_v, dst_v, buf0, buf1, buf2, zero_v, acc,
          semi, sg0, sg1, sg2):
        c = lax.axis_index("c")
        s = lax.axis_index("s")
        w = _worker(c, s)
        bufs = (buf0, buf1, buf2)
        semg = (sg0, sg1, sg2)

        idx_cps = [pltpu.async_copy(
            edge_hbm.at[0, pl.ds(w * EW, EW)], src_v, semi)]
        for j in range(NSEG):
            idx_cps.append(pltpu.async_copy(
                edge_hbm.at[1, pl.ds(w * EW + j * SEG, SEG)],
                dst_v.at[j], semi))

        @pl.when(c == 0)
        def _():
            pltpu.sync_copy(hs_hbm.at[pl.ds(s * SLC, SLC), :],
                            buf0.at[pl.ds(0, SLC), :])
            pltpu.sync_copy(buf0.at[pl.ds(0, SLC), :],
                            acc.at[pl.ds(s * SLC, SLC), :])

        @pl.when(c != 0)
        def _():
            def fillz(i, _):
                zero_v[i, :] = jnp.zeros((H,), jnp.float32)
                return 0
            lax.fori_loop(0, 64, fillz, 0)

            def zrow(i, _):
                pltpu.sync_copy(zero_v,
                                acc.at[pl.ds(s * SLC + i * 64, 64), :])
                return 0
            lax.fori_loop(0, SLC // 64, zrow, 0)

        for cp in idx_cps:
            cp.wait()
        plsc.subcore_barrier()

        gat = [None] * NSEG
        for j in range(NBUF):
            gat[j] = pltpu.async_copy(
                hs_hbm.at[src_v.at[pl.ds(j * SEG, SEG)]], bufs[j], semg[j])
        for j in range(NSEG):
            b = j % NBUF
            gat[j].wait()
            pltpu.sync_copy(bufs[b], acc.at[dst_v.at[j]], add=True)
            if j + NBUF < NSEG:
                gat[j + NBUF] = pltpu.async_copy(
                    hs_hbm.at[src_v.at[pl.ds((j + NBUF) * SEG, SEG)]],
                    bufs[b], semg[b])

        plsc.subcore_barrier()
        pltpu.sync_copy(acc.at[pl.ds(s * SLC, SLC), :],
                        out_hbm.at[c, pl.ds(s * SLC, SLC), :])

    return k(edge_index, hs)


def _sc_agg1(edge_index, gs):
    @functools.partial(
        pl.kernel,
        out_type=jax.ShapeDtypeStruct((NC, NP), jnp.float32),
        mesh=_mesh(),
        compiler_params=_SC_PARAMS,
        scratch_types=[
            pltpu.VMEM((EW,), jnp.int32),
            pltpu.VMEM((EW,), jnp.int32),
            pltpu.VMEM((EW,), jnp.float32),
            pltpu.VMEM((N,), jnp.float32),
            pltpu.VMEM((SLC,), jnp.float32),
            pltpu.VMEM_SHARED((NP,), jnp.float32),
            pltpu.SemaphoreType.DMA,
        ],
    )
    def k(edge_hbm, gs_hbm, out_hbm,
          src_v, dst_v, msg_v, gs_v, zero_v, acc, semi):
        c = lax.axis_index("c")
        s = lax.axis_index("s")
        w = _worker(c, s)

        cps = [
            pltpu.async_copy(edge_hbm.at[0, pl.ds(w * EW, EW)], src_v, semi),
            pltpu.async_copy(edge_hbm.at[1, pl.ds(w * EW, EW)], dst_v, semi),
            pltpu.async_copy(gs_hbm.at[pl.ds(0, N)], gs_v, semi),
        ]

        @pl.when(c == 0)
        def _():
            pltpu.sync_copy(gs_hbm.at[pl.ds(s * SLC, SLC)],
                            msg_v.at[pl.ds(0, SLC)])
            pltpu.sync_copy(msg_v.at[pl.ds(0, SLC)],
                            acc.at[pl.ds(s * SLC, SLC)])

        @pl.when(c != 0)
        def _():
            def fillz(i, _):
                zero_v[pl.ds(i * L, L)] = jnp.zeros((L,), jnp.float32)
                return 0
            lax.fori_loop(0, SLC // L, fillz, 0)
            pltpu.sync_copy(zero_v, acc.at[pl.ds(s * SLC, SLC)])

        for cp in cps:
            cp.wait()
        plsc.subcore_barrier()

        def gat(i, _):
            idx = src_v[pl.ds(i * L, L)]
            msg_v[pl.ds(i * L, L)] = plsc.load_gather(gs_v, [idx])
            return 0
        lax.fori_loop(0, EW // L, gat, 0)

        pltpu.sync_copy(msg_v, acc.at[dst_v], add=True)

        plsc.subcore_barrier()
        pltpu.sync_copy(acc.at[pl.ds(s * SLC, SLC)],
                        out_hbm.at[c, pl.ds(s * SLC, SLC)])

    return k(edge_index, gs)


def _tc_prep_body(cnt_ref, x_ref, w1_ref, dis_ref, hs_ref):
    deg = cnt_ref[0:1, :] + cnt_ref[1:2, :]
    dis = jnp.transpose(lax.rsqrt(deg))
    h = jnp.dot(x_ref[...], w1_ref[...], preferred_element_type=jnp.float32)
    dis_ref[...] = dis
    hs_ref[pl.ds(0, N), :] = h * dis[:N, :]
    hs_ref[pl.ds(N, NP - N), :] = jnp.zeros((NP - N, H), jnp.float32)


def _tc_prep(cnt, x, W1):
    return pl.pallas_call(
        _tc_prep_body,
        out_shape=[
            jax.ShapeDtypeStruct((NP, 1), jnp.float32),
            jax.ShapeDtypeStruct((NP, H), jnp.float32),
        ],
    )(cnt, x, W1)


def _tc_mid_body(agg_ref, dis_ref, b1_ref, w2_ref, gs_ref):
    a = agg_ref[0, :N, :] + agg_ref[1, :N, :]
    dis = dis_ref[:N, :]
    t = jnp.maximum(dis * a + b1_ref[...], 0.0)
    g = jnp.sum(t * w2_ref[...], axis=1, keepdims=True)
    gs_ref[pl.ds(0, N), :] = g * dis
    gs_ref[pl.ds(N, NP - N), :] = jnp.zeros((NP - N, 1), jnp.float32)


def _tc_mid(agg, dis, b1row, w2row):
    return pl.pallas_call(
        _tc_mid_body,
        out_shape=jax.ShapeDtypeStruct((NP, 1), jnp.float32),
    )(agg, dis, b1row, w2row)


def _tc_out_body(a2_ref, dis_ref, batch_ref, b2_ref, out_ref):
    bsz = (batch_ref[0, N - 1] + 1).astype(jnp.float32)
    a = a2_ref[0:1, :N] + a2_ref[1:2, :N]
    out_ref[...] = (dis_ref[0:1, :N] * a + b2_ref[0, 0]) * bsz


def _tc_out(a2, dis_row, batch_row, b2v):
    return pl.pallas_call(
        _tc_out_body,
        out_shape=jax.ShapeDtypeStruct((1, N), jnp.float32),
    )(a2, dis_row, batch_row, b2v)


def kernel(x, edge_index, batch, W1, b1, W2, b2):
    count = _sc_count(edge_index)
    dis, hs1 = _tc_prep(count, x, W1)

    agg1 = _sc_agg16(edge_index, hs1)
    gs = _tc_mid(agg1, dis,
                 b1.reshape(1, H), W2.reshape(1, H))

    agg2 = _sc_agg1(edge_index, gs.reshape(NP))
    out = _tc_out(agg2, dis.reshape(1, NP),
                  batch.reshape(1, N), b2.reshape(1, 1))
    return out


# speedup vs baseline: 124.1425x; 1.2025x over previous
import functools

import jax
import jax.numpy as jnp
from jax import lax
from jax.experimental import pallas as pl
from jax.experimental.pallas import tpu as pltpu
from jax.experimental.pallas import tpu_sc as plsc

N = 10000
E = 320000
F_IN = 128
H = 16

NC, NS, L = 2, 16, 16
NW = NC * NS
EW = E // NW
NSEG = 10
SEG = EW // NSEG
NBUF = 3
NP = 10240
SLC = NP // NS


def _mesh():
    return plsc.VectorSubcoreMesh(
        core_axis_name="c", subcore_axis_name="s",
        num_cores=NC, num_subcores=NS)


_SC_PARAMS = pltpu.CompilerParams(use_tc_tiling_on_sc=False,
                                  needs_layout_passes=False)


def _worker(c, s):
    return s * NC + c


def _sc_count(edge_index):
    @functools.partial(
        pl.kernel,
        out_type=jax.ShapeDtypeStruct((NC, NP), jnp.float32),
        mesh=_mesh(),
        compiler_params=_SC_PARAMS,
        scratch_types=[
            pltpu.VMEM((EW,), jnp.int32),
            pltpu.VMEM((EW,), jnp.float32),
            pltpu.VMEM((SLC,), jnp.float32),
            pltpu.VMEM_SHARED((NP,), jnp.float32),
            pltpu.SemaphoreType.DMA,
        ],
    )
    def k(edge_hbm, out_hbm, dst_v, ones_v, zero_v, acc, semi):
        c = lax.axis_index("c")
        s = lax.axis_index("s")
        w = _worker(c, s)

        cp = pltpu.async_copy(edge_hbm.at[1, pl.ds(w * EW, EW)], dst_v, semi)

        def fillo(i, _):
            ones_v[pl.ds(i * L, L)] = jnp.ones((L,), jnp.float32)
            return 0
        lax.fori_loop(0, EW // L, fillo, 0)

        def fillz(i, _):
            zero_v[pl.ds(i * L, L)] = jnp.zeros((L,), jnp.float32)
            return 0
        lax.fori_loop(0, SLC // L, fillz, 0)

        @pl.when(c == 0)
        def _():
            pltpu.sync_copy(ones_v.at[pl.ds(0, SLC)],
                            acc.at[pl.ds(s * SLC, SLC)])

        @pl.when(c != 0)
        def _():
            pltpu.sync_copy(zero_v, acc.at[pl.ds(s * SLC, SLC)])

        cp.wait()
        plsc.subcore_barrier()

        pltpu.sync_copy(ones_v, acc.at[dst_v], add=True)

        plsc.subcore_barrier()
        pltpu.sync_copy(acc.at[pl.ds(s * SLC, SLC)],
                        out_hbm.at[c, pl.ds(s * SLC, SLC)])

    return k(edge_index)


def _sc_agg16(edge_index, hs):
    @functools.partial(
        pl.kernel,
        out_type=jax.ShapeDtypeStruct((NC, NP, H), jnp.float32),
        mesh=_mesh(),
        compiler_params=_SC_PARAMS,
        scratch_types=[
            pltpu.VMEM((EW,), jnp.int32),
            pltpu.VMEM((NSEG, SEG), jnp.int32),
            pltpu.VMEM((SEG, H), jnp.float32),
            pltpu.VMEM((SEG, H), jnp.float32),
            pltpu.VMEM((SEG, H), jnp.float32),
            pltpu.VMEM((64, H), jnp.float32),
            pltpu.VMEM_SHARED((NP, H), jnp.float32),
            pltpu.SemaphoreType.DMA,
            pltpu.SemaphoreType.DMA,
            pltpu.SemaphoreType.DMA,
            pltpu.SemaphoreType.DMA,
        ],
    )
    def k(edge_hbm, hs_hbm, out_hbm,
          src_v, dst_v, buf0, buf1, buf2, zero_v, acc,
          semi, sg0, sg1, sg2):
        c = lax.axis_index("c")
        s = lax.axis_index("s")
        w = _worker(c, s)
        bufs = (buf0, buf1, buf2)
        semg = (sg0, sg1, sg2)

        idx_cps = [pltpu.async_copy(
            edge_hbm.at[0, pl.ds(w * EW, EW)], src_v, semi)]
        for j in range(NSEG):
            idx_cps.append(pltpu.async_copy(
                edge_hbm.at[1, pl.ds(w * EW + j * SEG, SEG)],
                dst_v.at[j], semi))

        @pl.when(c == 0)
        def _():
            pltpu.sync_copy(hs_hbm.at[pl.ds(s * SLC, SLC), :],
                            buf0.at[pl.ds(0, SLC), :])
            pltpu.sync_copy(buf0.at[pl.ds(0, SLC), :],
                            acc.at[pl.ds(s * SLC, SLC), :])

        @pl.when(c != 0)
        def _():
            def fillz(i, _):
                zero_v[i, :] = jnp.zeros((H,), jnp.float32)
                return 0
            lax.fori_loop(0, 64, fillz, 0)

            def zrow(i, _):
                pltpu.sync_copy(zero_v,
                                acc.at[pl.ds(s * SLC + i * 64, 64), :])
                return 0
            lax.fori_loop(0, SLC // 64, zrow, 0)

        for cp in idx_cps:
            cp.wait()
        plsc.subcore_barrier()

        gat = [None] * NSEG
        for j in range(NBUF):
            gat[j] = pltpu.async_copy(
                hs_hbm.at[src_v.at[pl.ds(j * SEG, SEG)]], bufs[j], semg[j])
        for j in range(NSEG):
            b = j % NBUF
            gat[j].wait()
            pltpu.sync_copy(bufs[b], acc.at[dst_v.at[j]], add=True)
            if j + NBUF < NSEG:
                gat[j + NBUF] = pltpu.async_copy(
                    hs_hbm.at[src_v.at[pl.ds((j + NBUF) * SEG, SEG)]],
                    bufs[b], semg[b])

        plsc.subcore_barrier()
        pltpu.sync_copy(acc.at[pl.ds(s * SLC, SLC), :],
                        out_hbm.at[c, pl.ds(s * SLC, SLC), :])

    return k(edge_index, hs)


def _sc_agg1(edge_index, gs):
    @functools.partial(
        pl.kernel,
        out_type=jax.ShapeDtypeStruct((NC, NP), jnp.float32),
        mesh=_mesh(),
        compiler_params=_SC_PARAMS,
        scratch_types=[
            pltpu.VMEM((EW,), jnp.int32),
            pltpu.VMEM((EW,), jnp.int32),
            pltpu.VMEM((EW,), jnp.float32),
            pltpu.VMEM((N,), jnp.float32),
            pltpu.VMEM((SLC,), jnp.float32),
            pltpu.VMEM_SHARED((NP,), jnp.float32),
            pltpu.SemaphoreType.DMA,
        ],
    )
    def k(edge_hbm, gs_hbm, out_hbm,
          src_v, dst_v, msg_v, gs_v, zero_v, acc, semi):
        c = lax.axis_index("c")
        s = lax.axis_index("s")
        w = _worker(c, s)

        cps = [
            pltpu.async_copy(edge_hbm.at[0, pl.ds(w * EW, EW)], src_v, semi),
            pltpu.async_copy(edge_hbm.at[1, pl.ds(w * EW, EW)], dst_v, semi),
            pltpu.async_copy(gs_hbm.at[pl.ds(0, N)], gs_v, semi),
        ]

        @pl.when(c == 0)
        def _():
            pltpu.sync_copy(gs_hbm.at[pl.ds(s * SLC, SLC)],
                            msg_v.at[pl.ds(0, SLC)])
            pltpu.sync_copy(msg_v.at[pl.ds(0, SLC)],
                            acc.at[pl.ds(s * SLC, SLC)])

        @pl.when(c != 0)
        def _():
            def fillz(i, _):
                zero_v[pl.ds(i * L, L)] = jnp.zeros((L,), jnp.float32)
                return 0
            lax.fori_loop(0, SLC // L, fillz, 0)
            pltpu.sync_copy(zero_v, acc.at[pl.ds(s * SLC, SLC)])

        for cp in cps:
            cp.wait()
        plsc.subcore_barrier()

        def gat(i, _):
            idx = src_v[pl.ds(i * L, L)]
            msg_v[pl.ds(i * L, L)] = plsc.load_gather(gs_v, [idx])
            return 0
        lax.fori_loop(0, EW // L, gat, 0)

        pltpu.sync_copy(msg_v, acc.at[dst_v], add=True)

        plsc.subcore_barrier()
        pltpu.sync_copy(acc.at[pl.ds(s * SLC, SLC)],
                        out_hbm.at[c, pl.ds(s * SLC, SLC)])

    return k(edge_index, gs)


def _tc_prep_body(cnt_ref, cnt80_ref, x_ref, w1_ref, dis80_ref, hs_ref):
    deg = cnt_ref[0:1, :] + cnt_ref[1:2, :]
    dis = jnp.transpose(lax.rsqrt(deg))
    h = jnp.dot(x_ref[...], w1_ref[...], preferred_element_type=jnp.float32)
    dis80_ref[...] = lax.rsqrt(cnt80_ref[0] + cnt80_ref[1])
    hs_ref[pl.ds(0, N), :] = h * dis[:N, :]
    hs_ref[pl.ds(N, NP - N), :] = jnp.zeros((NP - N, H), jnp.float32)


def _tc_prep(cnt, cnt80, x, W1):
    return pl.pallas_call(
        _tc_prep_body,
        out_shape=[
            jax.ShapeDtypeStruct((80, 128), jnp.float32),
            jax.ShapeDtypeStruct((NP, H), jnp.float32),
        ],
    )(cnt, cnt80, x, W1)


def _tc_mid_body(aggw_ref, dis80_ref, b1_ref, w2_ref, gs80_ref):
    lane2k = lax.broadcasted_iota(jnp.int32, (128, 16 * 128), 1) // H
    exp = jnp.where(lane2k == lax.broadcasted_iota(
        jnp.int32, (128, 16 * 128), 0), 1.0, 0.0)
    feat = lax.broadcasted_iota(jnp.int32, (H, 16 * 128), 1) % H
    sel = jnp.where(feat == lax.broadcasted_iota(
        jnp.int32, (H, 16 * 128), 0), 1.0, 0.0)
    red = jnp.where(lax.broadcasted_iota(
        jnp.int32, (16 * 128, 128), 0) // H == lax.broadcasted_iota(
        jnp.int32, (16 * 128, 128), 1), 1.0, 0.0)

    dis80 = dis80_ref[...]
    dis_ex = jnp.dot(dis80, exp, preferred_element_type=jnp.float32)
    b1_ex = jnp.dot(b1_ref[...], sel, preferred_element_type=jnp.float32)
    w2_ex = jnp.dot(w2_ref[...], sel, preferred_element_type=jnp.float32)
    a = aggw_ref[0] + aggw_ref[1]
    t = jnp.maximum(dis_ex * a + b1_ex, 0.0)
    g80 = jnp.dot(t * w2_ex, red, preferred_element_type=jnp.float32)
    gs80_ref[...] = g80 * dis80


def _tc_mid(aggw, dis80, b1row, w2row):
    return pl.pallas_call(
        _tc_mid_body,
        out_shape=jax.ShapeDtypeStruct((80, 128), jnp.float32),
    )(aggw, dis80, b1row, w2row)


def _tc_out_body(a2w_ref, dis80_ref, batch_ref, b2_ref, out_ref):
    bsz = (batch_ref[N - 1] + 1).astype(jnp.float32)
    a = a2w_ref[0] + a2w_ref[1]
    out_ref[...] = (dis80_ref[...] * a + b2_ref[0, 0]) * bsz


def _tc_out(a2w, dis80, batch, b2v):
    return pl.pallas_call(
        _tc_out_body,
        out_shape=jax.ShapeDtypeStruct((80, 128), jnp.float32),
    )(a2w, dis80, batch, b2v)


def kernel(x, edge_index, batch, W1, b1, W2, b2):
    count = _sc_count(edge_index)
    dis80, hs1 = _tc_prep(count, count.reshape(2, 80, 128), x, W1)

    agg1 = _sc_agg16(edge_index, hs1)
    gs80 = _tc_mid(agg1.reshape(2, 80, 16 * 128), dis80,
                   b1.reshape(1, H), W2.reshape(1, H))

    agg2 = _sc_agg1(edge_index, gs80.reshape(NP))
    out80 = _tc_out(agg2.reshape(2, 80, 128), dis80,
                    batch, b2.reshape(1, 1))
    return out80.reshape(NP)[:N].reshape(1, N)


# speedup vs baseline: 124.2677x; 1.0010x over previous
import functools

import jax
import jax.numpy as jnp
from jax import lax
from jax.experimental import pallas as pl
from jax.experimental.pallas import tpu as pltpu
from jax.experimental.pallas import tpu_sc as plsc

N = 10000
E = 320000
F_IN = 128
H = 16

NC, NS, L = 2, 16, 16
NW = NC * NS
EW = E // NW
NSEG = 10
SEG = EW // NSEG
NBUF = 3
NP = 10240
SLC = NP // NS


def _mesh():
    return plsc.VectorSubcoreMesh(
        core_axis_name="c", subcore_axis_name="s",
        num_cores=NC, num_subcores=NS)


_SC_PARAMS = pltpu.CompilerParams(use_tc_tiling_on_sc=False,
                                  needs_layout_passes=False)


def _worker(c, s):
    return s * NC + c


def _sc_count(edge_index):
    @functools.partial(
        pl.kernel,
        out_type=jax.ShapeDtypeStruct((NC, NP), jnp.float32),
        mesh=_mesh(),
        compiler_params=_SC_PARAMS,
        scratch_types=[
            pltpu.VMEM((EW,), jnp.int32),
            pltpu.VMEM((EW,), jnp.float32),
            pltpu.VMEM((SLC,), jnp.float32),
            pltpu.VMEM_SHARED((NP,), jnp.float32),
            pltpu.SemaphoreType.DMA,
        ],
    )
    def k(edge_hbm, out_hbm, dst_v, ones_v, zero_v, acc, semi):
        c = lax.axis_index("c")
        s = lax.axis_index("s")
        w = _worker(c, s)

        cp = pltpu.async_copy(edge_hbm.at[1, pl.ds(w * EW, EW)], dst_v, semi)

        def fillo(i, _):
            ones_v[pl.ds(i * L, L)] = jnp.ones((L,), jnp.float32)
            return 0
        lax.fori_loop(0, EW // L, fillo, 0)

        def fillz(i, _):
            zero_v[pl.ds(i * L, L)] = jnp.zeros((L,), jnp.float32)
            return 0
        lax.fori_loop(0, SLC // L, fillz, 0)

        @pl.when(c == 0)
        def _():
            pltpu.sync_copy(ones_v.at[pl.ds(0, SLC)],
                            acc.at[pl.ds(s * SLC, SLC)])

        @pl.when(c != 0)
        def _():
            pltpu.sync_copy(zero_v, acc.at[pl.ds(s * SLC, SLC)])

        cp.wait()
        plsc.subcore_barrier()

        pltpu.sync_copy(ones_v, acc.at[dst_v], add=True)

        plsc.subcore_barrier()
        pltpu.sync_copy(acc.at[pl.ds(s * SLC, SLC)],
                        out_hbm.at[c, pl.ds(s * SLC, SLC)])

    return k(edge_index)


def _sc_agg16(edge_index, hs):
    @functools.partial(
        pl.kernel,
        out_type=jax.ShapeDtypeStruct((NC, NP, H), jnp.float32),
        mesh=_mesh(),
        compiler_params=_SC_PARAMS,
        scratch_types=[
            pltpu.VMEM((EW,), jnp.int32),
            pltpu.VMEM((NSEG, SEG), jnp.int32),
            pltpu.VMEM((SEG, H), jnp.float32),
            pltpu.VMEM((SEG, H), jnp.float32),
            pltpu.VMEM((SEG, H), jnp.float32),
            pltpu.VMEM((64, H), jnp.float32),
            pltpu.VMEM_SHARED((NP, H), jnp.float32),
            pltpu.SemaphoreType.DMA,
            pltpu.SemaphoreType.DMA,
            pltpu.SemaphoreType.DMA,
            pltpu.SemaphoreType.DMA,
        ],
    )
    def k(edge_hbm, hs_hbm, out_hbm,
          src_v, dst_v, buf0, buf1, buf2, zero_v, acc,
          semi, sg0, sg1, sg2):
        c = lax.axis_index("c")
        s = lax.axis_index("s")
        w = _worker(c, s)
        bufs = (buf0, buf1, buf2)
        semg = (sg0, sg1, sg2)

        idx_cps = [pltpu.async_copy(
            edge_hbm.at[0, pl.ds(w * EW, EW)], src_v, semi)]
        for j in range(NSEG):
            idx_cps.append(pltpu.async_copy(
                edge_hbm.at[1, pl.ds(w * EW + j * SEG, SEG)],
                dst_v.at[j], semi))

        @pl.when(c == 0)
        def _():
            pltpu.sync_copy(hs_hbm.at[pl.ds(s * SLC, SLC), :],
                            buf0.at[pl.ds(0, SLC), :])
            pltpu.sync_copy(buf0.at[pl.ds(0, SLC), :],
                            acc.at[pl.ds(s * SLC, SLC), :])

        @pl.when(c != 0)
        def _():
            def fillz(i, _):
                zero_v[i, :] = jnp.zeros((H,), jnp.float32)
                return 0
            lax.fori_loop(0, 64, fillz, 0)

            def zrow(i, _):
                pltpu.sync_copy(zero_v,
                                acc.at[pl.ds(s * SLC + i * 64, 64), :])
                return 0
            lax.fori_loop(0, SLC // 64, zrow, 0)

        for cp in idx_cps:
            cp.wait()
        plsc.subcore_barrier()

        gat = [None] * NSEG
        for j in range(NBUF):
            gat[j] = pltpu.async_copy(
                hs_hbm.at[src_v.at[pl.ds(j * SEG, SEG)]], bufs[j], semg[j])
        for j in range(NSEG):
            b = j % NBUF
            gat[j].wait()
            pltpu.sync_copy(bufs[b], acc.at[dst_v.at[j]], add=True)
            if j + NBUF < NSEG:
                gat[j + NBUF] = pltpu.async_copy(
                    hs_hbm.at[src_v.at[pl.ds((j + NBUF) * SEG, SEG)]],
                    bufs[b], semg[b])

        plsc.subcore_barrier()
        pltpu.sync_copy(acc.at[pl.ds(s * SLC, SLC), :],
                        out_hbm.at[c, pl.ds(s * SLC, SLC), :])

    return k(edge_index, hs)


def _sc_agg1(edge_index, gs):
    @functools.partial(
        pl.kernel,
        out_type=jax.ShapeDtypeStruct((NC, NP), jnp.float32),
        mesh=_mesh(),
        compiler_params=_SC_PARAMS,
        scratch_types=[
            pltpu.VMEM((EW,), jnp.int32),
            pltpu.VMEM((EW,), jnp.int32),
            pltpu.VMEM((EW,), jnp.float32),
            pltpu.VMEM((N,), jnp.float32),
            pltpu.VMEM((SLC,), jnp.float32),
            pltpu.VMEM_SHARED((NP,), jnp.float32),
            pltpu.SemaphoreType.DMA,
        ],
    )
    def k(edge_hbm, gs_hbm, out_hbm,
          src_v, dst_v, msg_v, gs_v, zero_v, acc, semi):
        c = lax.axis_index("c")
        s = lax.axis_index("s")
        w = _worker(c, s)

        cps = [
            pltpu.async_copy(edge_hbm.at[0, pl.ds(w * EW, EW)], src_v, semi),
            pltpu.async_copy(edge_hbm.at[1, pl.ds(w * EW, EW)], dst_v, semi),
            pltpu.async_copy(gs_hbm.at[pl.ds(0, N)], gs_v, semi),
        ]

        @pl.when(c == 0)
        def _():
            pltpu.sync_copy(gs_hbm.at[pl.ds(s * SLC, SLC)],
                            msg_v.at[pl.ds(0, SLC)])
            pltpu.sync_copy(msg_v.at[pl.ds(0, SLC)],
                            acc.at[pl.ds(s * SLC, SLC)])

        @pl.when(c != 0)
        def _():
            def fillz(i, _):
                zero_v[pl.ds(i * L, L)] = jnp.zeros((L,), jnp.float32)
                return 0
            lax.fori_loop(0, SLC // L, fillz, 0)
            pltpu.sync_copy(zero_v, acc.at[pl.ds(s * SLC, SLC)])

        for cp in cps:
            cp.wait()
        plsc.subcore_barrier()

        def gat(i, _):
            idx = src_v[pl.ds(i * L, L)]
            msg_v[pl.ds(i * L, L)] = plsc.load_gather(gs_v, [idx])
            return 0
        lax.fori_loop(0, EW // L, gat, 0)

        pltpu.sync_copy(msg_v, acc.at[dst_v], add=True)

        plsc.subcore_barrier()
        pltpu.sync_copy(acc.at[pl.ds(s * SLC, SLC)],
                        out_hbm.at[c, pl.ds(s * SLC, SLC)])

    return k(edge_index, gs)


def _tc_prep_body(cnt_ref, cnt80_ref, x_ref, w1_ref, dis80_ref, hs_ref):
    deg = cnt_ref[0:1, :] + cnt_ref[1:2, :]
    dis = jnp.transpose(lax.rsqrt(deg))
    h = jnp.dot(x_ref[...], w1_ref[...], preferred_element_type=jnp.float32)
    dis80_ref[...] = lax.rsqrt(cnt80_ref[0] + cnt80_ref[1])
    hs_ref[pl.ds(0, N), :] = h * dis[:N, :]
    hs_ref[pl.ds(N, NP - N), :] = jnp.zeros((NP - N, H), jnp.float32)


def _tc_prep(cnt, cnt80, x, W1):
    return pl.pallas_call(
        _tc_prep_body,
        out_shape=[
            jax.ShapeDtypeStruct((80, 128), jnp.float32),
            jax.ShapeDtypeStruct((NP, H), jnp.float32),
        ],
    )(cnt, cnt80, x, W1)


def _tc_mid_body(aggw_ref, dis80_ref, b1_ref, w2_ref, gs80_ref):
    lane2k = lax.broadcasted_iota(jnp.int32, (128, 16 * 128), 1) // H
    exp = jnp.where(lane2k == lax.broadcasted_iota(
        jnp.int32, (128, 16 * 128), 0), 1.0, 0.0)
    feat = lax.broadcasted_iota(jnp.int32, (H, 16 * 128), 1) % H
    sel = jnp.where(feat == lax.broadcasted_iota(
        jnp.int32, (H, 16 * 128), 0), 1.0, 0.0)
    red = jnp.where(lax.broadcasted_iota(
        jnp.int32, (16 * 128, 128), 0) // H == lax.broadcasted_iota(
        jnp.int32, (16 * 128, 128), 1), 1.0, 0.0)

    dis80 = dis80_ref[...]
    dis_ex = jnp.dot(dis80, exp, preferred_element_type=jnp.float32)
    b1_ex = jnp.dot(b1_ref[...], sel, preferred_element_type=jnp.float32)
    w2_ex = jnp.dot(w2_ref[...], sel, preferred_element_type=jnp.float32)
    a = aggw_ref[0] + aggw_ref[1]
    t = jnp.maximum(dis_ex * a + b1_ex, 0.0)
    g80 = jnp.dot(t * w2_ex, red, preferred_element_type=jnp.float32)
    gs80_ref[...] = g80 * dis80


def _tc_mid(aggw, dis80, b1row, w2row):
    return pl.pallas_call(
        _tc_mid_body,
        out_shape=jax.ShapeDtypeStruct((80, 128), jnp.float32),
    )(aggw, dis80, b1row, w2row)


def _tc_out_body(a2w_ref, dis80_ref, bsz_ref, b2_ref, out_ref):
    a = a2w_ref[0] + a2w_ref[1]
    out_ref[...] = (dis80_ref[...] * a + b2_ref[0, 0]) * bsz_ref[0, 0]


def _tc_out(a2w, dis80, bszv, b2v):
    return pl.pallas_call(
        _tc_out_body,
        out_shape=jax.ShapeDtypeStruct((80, 128), jnp.float32),
    )(a2w, dis80, bszv, b2v)


def kernel(x, edge_index, batch, W1, b1, W2, b2):
    count = _sc_count(edge_index)
    dis80, hs1 = _tc_prep(count, count.reshape(2, 80, 128), x, W1)

    agg1 = _sc_agg16(edge_index, hs1)
    gs80 = _tc_mid(agg1.reshape(2, 80, 16 * 128), dis80,
                   b1.reshape(1, H), W2.reshape(1, H))

    agg2 = _sc_agg1(edge_index, gs80.reshape(NP))
    bszv = (batch[-1] + 1).astype(jnp.float32).reshape(1, 1)
    out80 = _tc_out(agg2.reshape(2, 80, 128), dis80,
                    bszv, b2.reshape(1, 1))
    return out80.reshape(NP)[:N].reshape(1, N)
